# baseline (device time: 290612 ns/iter reference)
import jax
import jax.numpy as jnp
from jax import lax
from jax.experimental import pallas as pl
from jax.experimental.pallas import tpu as pltpu

N_DEV = 4
B, S, D = 2, 512, 2048
H, Dh, Dr = 16, 128, 32


def _ring_allreduce(kv):
    rows, cols = kv.shape
    chunk = rows // N_DEV

    def body(x_ref, out_ref, rbuf, rs_send, rs_recv, ag_send, ag_recv):
        me = lax.axis_index("i")
        left = (me + N_DEV - 1) % N_DEV
        right = (me + 1) % N_DEV

        barrier = pltpu.get_barrier_semaphore()
        for nbr in (left, right):
            pl.semaphore_signal(barrier, inc=1, device_id=(nbr,),
                                device_id_type=pl.DeviceIdType.MESH)
        pl.semaphore_wait(barrier, 2)

        out_ref[...] = x_ref[...]

        for s in range(N_DEV - 1):
            send_c = (me + N_DEV - s) % N_DEV
            rdma = pltpu.make_async_remote_copy(
                src_ref=out_ref.at[pl.ds(send_c * chunk, chunk), :],
                dst_ref=rbuf.at[s],
                send_sem=rs_send.at[s],
                recv_sem=rs_recv.at[s],
                device_id=(right,),
                device_id_type=pl.DeviceIdType.MESH,
            )
            rdma.start()
            rdma.wait()
            recv_c = (me + 2 * N_DEV - 1 - s) % N_DEV
            out_ref[pl.ds(recv_c * chunk, chunk), :] += rbuf[s]

        for nbr in (left, right):
            pl.semaphore_signal(barrier, inc=1, device_id=(nbr,),
                                device_id_type=pl.DeviceIdType.MESH)
        pl.semaphore_wait(barrier, 2)

        for s in range(N_DEV - 1):
            send_c = (me + 1 + N_DEV - s) % N_DEV
            rdma = pltpu.make_async_remote_copy(
                src_ref=out_ref.at[pl.ds(send_c * chunk, chunk), :],
                dst_ref=out_ref.at[pl.ds(send_c * chunk, chunk), :],
                send_sem=ag_send.at[s],
                recv_sem=ag_recv.at[s],
                device_id=(right,),
                device_id_type=pl.DeviceIdType.MESH,
            )
            rdma.start()
            rdma.wait()

    return pl.pallas_call(
        body,
        out_shape=jax.ShapeDtypeStruct((rows, cols), kv.dtype),
        in_specs=[pl.BlockSpec(memory_space=pltpu.VMEM)],
        out_specs=pl.BlockSpec(memory_space=pltpu.VMEM),
        scratch_shapes=[
            pltpu.VMEM((N_DEV - 1, chunk, cols), kv.dtype),
            pltpu.SemaphoreType.DMA((N_DEV - 1,)),
            pltpu.SemaphoreType.DMA((N_DEV - 1,)),
            pltpu.SemaphoreType.DMA((N_DEV - 1,)),
            pltpu.SemaphoreType.DMA((N_DEV - 1,)),
        ],
        compiler_params=pltpu.CompilerParams(collective_id=0),
    )(kv)


def kernel(x, Wdkv, Wuk, Wuv, Wq, Wqr, Wkr, Wo):
    f = jnp.bfloat16
    x2 = x.astype(f).reshape(B * S, D)
    c = x2 @ Wdkv.astype(f)
    kv_part = jnp.concatenate(
        [c @ Wuk.astype(f), c @ Wuv.astype(f)], axis=-1)
    kv = _ring_allreduce(kv_part)
    K = kv[:, :D].reshape(B, S, H, Dh)
    V = kv[:, D:].reshape(B, S, H, Dh)
    Q = (x2 @ Wq.astype(f)).reshape(B, S, H, Dh)
    Qr = (x2 @ Wqr.astype(f)).reshape(B, S, H, Dr)
    Kr = (x2 @ Wkr.astype(f)).reshape(B, S, Dr)
    scale = (Dh + Dr) ** -0.5
    scores = jnp.einsum("bshd,bthd->bhst", Q, K,
                        preferred_element_type=jnp.float32)
    scores = scores + jnp.einsum("bshr,btr->bhst", Qr, Kr,
                                 preferred_element_type=jnp.float32)
    P = jax.nn.softmax(scores * scale, axis=-1).astype(f)
    O = jnp.einsum("bhst,bthd->bshd", P, V,
                   preferred_element_type=jnp.float32)
    out = (O.reshape(B * S, H * Dh).astype(f) @ Wo.astype(f))
    return out.astype(jnp.float32).reshape(B, S, D)


# device time: 126557 ns/iter; 2.2963x vs baseline; 2.2963x over previous
import jax
import jax.numpy as jnp
from jax import lax
from jax.experimental import pallas as pl
from jax.experimental.pallas import tpu as pltpu

N_DEV = 4
B, S, D = 2, 512, 2048
H, Dh, Dr = 16, 128, 32
HG = H // N_DEV
GC = HG * Dh


def _signal_all_and_wait(me):
    barrier = pltpu.get_barrier_semaphore()
    for k in range(1, N_DEV):
        pl.semaphore_signal(barrier, inc=1, device_id=((me + k) % N_DEV,),
                            device_id_type=pl.DeviceIdType.MESH)
    pl.semaphore_wait(barrier, N_DEV - 1)


def _reduce_scatter_groups(kvg):
    _, rows, cols = kvg.shape

    def body(kv_ref, out_ref, rbuf, send_sems, recv_sems):
        me = lax.axis_index("i")
        _signal_all_and_wait(me)
        rdmas = []
        for k in range(1, N_DEV):
            tgt = (me + k) % N_DEV
            slot = N_DEV - 1 - k
            rdma = pltpu.make_async_remote_copy(
                src_ref=kv_ref.at[tgt],
                dst_ref=rbuf.at[slot],
                send_sem=send_sems.at[slot],
                recv_sem=recv_sems.at[slot],
                device_id=(tgt,),
                device_id_type=pl.DeviceIdType.MESH,
            )
            rdma.start()
            rdmas.append(rdma)
        for rdma in rdmas:
            rdma.wait()
        acc = kv_ref[me].astype(jnp.float32)
        for slot in range(N_DEV - 1):
            acc = acc + rbuf[slot].astype(jnp.float32)
        out_ref[...] = acc.astype(kvg.dtype)

    return pl.pallas_call(
        body,
        out_shape=jax.ShapeDtypeStruct((rows, cols), kvg.dtype),
        in_specs=[pl.BlockSpec(memory_space=pltpu.VMEM)],
        out_specs=pl.BlockSpec(memory_space=pltpu.VMEM),
        scratch_shapes=[
            pltpu.VMEM((N_DEV - 1, rows, cols), kvg.dtype),
            pltpu.SemaphoreType.DMA((N_DEV - 1,)),
            pltpu.SemaphoreType.DMA((N_DEV - 1,)),
        ],
        compiler_params=pltpu.CompilerParams(collective_id=0),
    )(kvg)


def _allgather_groups(o_g):
    rows, gcols = o_g.shape

    def body(o_ref, out_ref, send_sems, recv_sems):
        me = lax.axis_index("i")
        _signal_all_and_wait(me)
        out_ref[:, pl.ds(me * gcols, gcols)] = o_ref[...]
        rdmas = []
        for k in range(1, N_DEV):
            tgt = (me + k) % N_DEV
            slot = N_DEV - 1 - k
            rdma = pltpu.make_async_remote_copy(
                src_ref=o_ref,
                dst_ref=out_ref.at[:, pl.ds(me * gcols, gcols)],
                send_sem=send_sems.at[slot],
                recv_sem=recv_sems.at[slot],
                device_id=(tgt,),
                device_id_type=pl.DeviceIdType.MESH,
            )
            rdma.start()
            rdmas.append(rdma)
        for rdma in rdmas:
            rdma.wait()

    return pl.pallas_call(
        body,
        out_shape=jax.ShapeDtypeStruct((rows, N_DEV * gcols), o_g.dtype),
        in_specs=[pl.BlockSpec(memory_space=pltpu.VMEM)],
        out_specs=pl.BlockSpec(memory_space=pltpu.VMEM),
        scratch_shapes=[
            pltpu.SemaphoreType.DMA((N_DEV - 1,)),
            pltpu.SemaphoreType.DMA((N_DEV - 1,)),
        ],
        compiler_params=pltpu.CompilerParams(collective_id=1),
    )(o_g)


def kernel(x, Wdkv, Wuk, Wuv, Wq, Wqr, Wkr, Wo):
    f = jnp.bfloat16
    me = lax.axis_index("i")
    x2 = x.astype(f).reshape(B * S, D)
    c = x2 @ Wdkv.astype(f)
    Kp = (c @ Wuk.astype(f)).reshape(B * S, N_DEV, GC)
    Vp = (c @ Wuv.astype(f)).reshape(B * S, N_DEV, GC)
    kvg = jnp.moveaxis(jnp.concatenate([Kp, Vp], axis=-1), 1, 0)
    block = _reduce_scatter_groups(kvg)

    Kg = block[:, :GC].reshape(B, S, HG, Dh)
    Vg = block[:, GC:].reshape(B, S, HG, Dh)
    Wq_g = lax.dynamic_slice(Wq, (0, me * GC), (D, GC)).astype(f)
    Wqr_g = lax.dynamic_slice(Wqr, (0, me * HG * Dr), (D, HG * Dr)).astype(f)
    Qg = (x2 @ Wq_g).reshape(B, S, HG, Dh)
    Qrg = (x2 @ Wqr_g).reshape(B, S, HG, Dr)
    Kr = (x2 @ Wkr.astype(f)).reshape(B, S, Dr)

    scale = (Dh + Dr) ** -0.5
    scores = jnp.einsum("bshd,bthd->bhst", Qg, Kg,
                        preferred_element_type=jnp.float32)
    scores = scores + jnp.einsum("bshr,btr->bhst", Qrg, Kr,
                                 preferred_element_type=jnp.float32)
    P = jax.nn.softmax(scores * scale, axis=-1).astype(f)
    Og = jnp.einsum("bhst,bthd->bshd", P, Vg,
                    preferred_element_type=jnp.float32)
    Og = Og.reshape(B * S, GC).astype(f)

    O_full = _allgather_groups(Og)
    out = O_full @ Wo.astype(f)
    return out.astype(jnp.float32).reshape(B, S, D)


# device time: 85382 ns/iter; 3.4037x vs baseline; 1.4822x over previous
import jax
import jax.numpy as jnp
from jax import lax
from jax.experimental import pallas as pl
from jax.experimental.pallas import tpu as pltpu

N_DEV = 4
B, S, D = 2, 512, 2048
H, Dh, Dr = 16, 128, 32
HG = H // N_DEV
GC = HG * Dh


def _signal_all_and_wait(me):
    barrier = pltpu.get_barrier_semaphore()
    for k in range(1, N_DEV):
        pl.semaphore_signal(barrier, inc=1, device_id=((me + k) % N_DEV,),
                            device_id_type=pl.DeviceIdType.MESH)
    pl.semaphore_wait(barrier, N_DEV - 1)


def _gather_c_and_weights(c_me, Wuk_s, Wuv_s):
    rows, dcs = c_me.shape

    def body(c_ref, wk_ref, wv_ref, cf_ref, wkg_ref, wvg_ref,
             send_sems, recv_sems):
        me = lax.axis_index("i")
        _signal_all_and_wait(me)
        cf_ref[:, pl.ds(me * dcs, dcs)] = c_ref[...]
        wkg_ref[pl.ds(me * dcs, dcs), :] = wk_ref[:, pl.ds(me * GC, GC)]
        wvg_ref[pl.ds(me * dcs, dcs), :] = wv_ref[:, pl.ds(me * GC, GC)]
        rdmas = []
        for k in range(1, N_DEV):
            tgt = (me + k) % N_DEV
            base = 3 * (N_DEV - 1 - k)
            for t, (src, dst) in enumerate([
                (c_ref, cf_ref.at[:, pl.ds(me * dcs, dcs)]),
                (wk_ref.at[:, pl.ds(tgt * GC, GC)],
                 wkg_ref.at[pl.ds(me * dcs, dcs), :]),
                (wv_ref.at[:, pl.ds(tgt * GC, GC)],
                 wvg_ref.at[pl.ds(me * dcs, dcs), :]),
            ]):
                rdma = pltpu.make_async_remote_copy(
                    src_ref=src,
                    dst_ref=dst,
                    send_sem=send_sems.at[base + t],
                    recv_sem=recv_sems.at[base + t],
                    device_id=(tgt,),
                    device_id_type=pl.DeviceIdType.MESH,
                )
                rdma.start()
                rdmas.append(rdma)
        for rdma in rdmas:
            rdma.wait()

    n_sems = 3 * (N_DEV - 1)
    return pl.pallas_call(
        body,
        out_shape=[
            jax.ShapeDtypeStruct((rows, N_DEV * dcs), c_me.dtype),
            jax.ShapeDtypeStruct((N_DEV * dcs, GC), c_me.dtype),
            jax.ShapeDtypeStruct((N_DEV * dcs, GC), c_me.dtype),
        ],
        in_specs=[pl.BlockSpec(memory_space=pltpu.VMEM)] * 3,
        out_specs=[pl.BlockSpec(memory_space=pltpu.VMEM)] * 3,
        scratch_shapes=[
            pltpu.SemaphoreType.DMA((n_sems,)),
            pltpu.SemaphoreType.DMA((n_sems,)),
        ],
        compiler_params=pltpu.CompilerParams(collective_id=0),
    )(c_me, Wuk_s, Wuv_s)


def _allgather_groups(o_g):
    rows, gcols = o_g.shape

    def body(o_ref, out_ref, send_sems, recv_sems):
        me = lax.axis_index("i")
        _signal_all_and_wait(me)
        out_ref[:, pl.ds(me * gcols, gcols)] = o_ref[...]
        rdmas = []
        for k in range(1, N_DEV):
            tgt = (me + k) % N_DEV
            slot = N_DEV - 1 - k
            rdma = pltpu.make_async_remote_copy(
                src_ref=o_ref,
                dst_ref=out_ref.at[:, pl.ds(me * gcols, gcols)],
                send_sem=send_sems.at[slot],
                recv_sem=recv_sems.at[slot],
                device_id=(tgt,),
                device_id_type=pl.DeviceIdType.MESH,
            )
            rdma.start()
            rdmas.append(rdma)
        for rdma in rdmas:
            rdma.wait()

    return pl.pallas_call(
        body,
        out_shape=jax.ShapeDtypeStruct((rows, N_DEV * gcols), o_g.dtype),
        in_specs=[pl.BlockSpec(memory_space=pltpu.VMEM)],
        out_specs=pl.BlockSpec(memory_space=pltpu.VMEM),
        scratch_shapes=[
            pltpu.SemaphoreType.DMA((N_DEV - 1,)),
            pltpu.SemaphoreType.DMA((N_DEV - 1,)),
        ],
        compiler_params=pltpu.CompilerParams(collective_id=1),
    )(o_g)


def kernel(x, Wdkv, Wuk, Wuv, Wq, Wqr, Wkr, Wo):
    f = jnp.bfloat16
    me = lax.axis_index("i")
    x2 = x.astype(f).reshape(B * S, D)
    c = x2 @ Wdkv.astype(f)
    c_full, Wk_g, Wv_g = _gather_c_and_weights(
        c, Wuk.astype(f), Wuv.astype(f))
    Kg = (c_full @ Wk_g).reshape(B, S, HG, Dh)
    Vg = (c_full @ Wv_g).reshape(B, S, HG, Dh)
    Wq_g = lax.dynamic_slice(Wq, (0, me * GC), (D, GC)).astype(f)
    Wqr_g = lax.dynamic_slice(Wqr, (0, me * HG * Dr), (D, HG * Dr)).astype(f)
    Qg = (x2 @ Wq_g).reshape(B, S, HG, Dh)
    Qrg = (x2 @ Wqr_g).reshape(B, S, HG, Dr)
    Kr = (x2 @ Wkr.astype(f)).reshape(B, S, Dr)

    scale = (Dh + Dr) ** -0.5
    scores = jnp.einsum("bshd,bthd->bhst", Qg, Kg,
                        preferred_element_type=jnp.float32)
    scores = scores + jnp.einsum("bshr,btr->bhst", Qrg, Kr,
                                 preferred_element_type=jnp.float32)
    P = jax.nn.softmax(scores * scale, axis=-1).astype(f)
    Og = jnp.einsum("bhst,bthd->bshd", P, Vg,
                    preferred_element_type=jnp.float32)
    Og = Og.reshape(B * S, GC).astype(f)

    O_full = _allgather_groups(Og)
    out = O_full @ Wo.astype(f)
    return out.astype(jnp.float32).reshape(B, S, D)
